# baseline (speedup 1.0000x reference)
import jax
import jax.numpy as jnp
from jax.experimental import pallas as pl
from jax.experimental.pallas import tpu as pltpu
from jax.experimental.pallas import tpu_sc as plsc

VOCAB = 100000
D = 64
U = 128
B = 1024
T = 50

GATHER_WINDOW = 256


def _sc_gather(table2, idx_flat):
    n = idx_flat.shape[0]
    idx2 = idx_flat.reshape(1, n)
    mesh = plsc.VectorSubcoreMesh(core_axis_name="c", subcore_axis_name="s")

    @pl.kernel(
        out_type=jax.ShapeDtypeStruct((n, 2 * D), table2.dtype),
        mesh=mesh,
    )
    def gather_kernel(tab_hbm, i_hbm, o_hbm):
        def body(i_vmem, o_vmem):
            pltpu.sync_copy(tab_hbm.at[i_vmem.at[0]], o_vmem)

        pltpu.emit_pipeline(
            body,
            grid=(n // GATHER_WINDOW,),
            in_specs=[pl.BlockSpec((1, GATHER_WINDOW), lambda i: (0, i))],
            out_specs=[pl.BlockSpec((GATHER_WINDOW, 2 * D), lambda i: (i, 0))],
            core_axis_name=("c", "s"),
            dimension_semantics=(pltpu.PARALLEL,),
        )(i_hbm, o_hbm)

    return gather_kernel(table2, idx2)


def _gru_body(hid_ref, emb_ref, par_ref, k_ref, rk_ref, bi_ref, br_ref,
              out_ref, state_ref, h_ref):
    t = pl.program_id(0)

    @pl.when(t == 0)
    def _():
        h_ref[...] = hid_ref[...]

    g = emb_ref[0]
    par = par_ref[0]
    xt = jnp.where(par > 0.5, g[:, D:], g[:, :D])
    h = h_ref[...]
    cx = jnp.dot(xt, k_ref[...], preferred_element_type=jnp.float32)
    cx = cx + bi_ref[...]
    ch = jnp.dot(h, rk_ref[...], preferred_element_type=jnp.float32)
    ch = ch + br_ref[...]
    z = jax.nn.sigmoid(cx[:, :U] + ch[:, :U])
    r = jax.nn.sigmoid(cx[:, U:2 * U] + ch[:, U:2 * U])
    hh = jnp.tanh(cx[:, 2 * U:] + r * ch[:, 2 * U:])
    h_new = z * h + (1.0 - z) * hh
    h_ref[...] = h_new
    out_ref[...] = h_new

    @pl.when(t == T - 1)
    def _():
        state_ref[...] = h_new


def _tc_gru(emb, parity, hidden, k, rk, bi, br):
    return pl.pallas_call(
        _gru_body,
        grid=(T,),
        in_specs=[
            pl.BlockSpec((B, U), lambda t: (0, 0)),
            pl.BlockSpec((1, B, 2 * D), lambda t: (t, 0, 0)),
            pl.BlockSpec((1, B, 1), lambda t: (t, 0, 0)),
            pl.BlockSpec((D, 3 * U), lambda t: (0, 0)),
            pl.BlockSpec((U, 3 * U), lambda t: (0, 0)),
            pl.BlockSpec((1, 3 * U), lambda t: (0, 0)),
            pl.BlockSpec((1, 3 * U), lambda t: (0, 0)),
        ],
        out_specs=[
            pl.BlockSpec((B, U), lambda t: (0, t)),
            pl.BlockSpec((B, U), lambda t: (0, 0)),
        ],
        out_shape=[
            jax.ShapeDtypeStruct((B, T * U), jnp.float32),
            jax.ShapeDtypeStruct((B, U), jnp.float32),
        ],
        scratch_shapes=[pltpu.VMEM((B, U), jnp.float32)],
        compiler_params=pltpu.CompilerParams(
            dimension_semantics=("arbitrary",),
        ),
    )(hidden, emb, parity, k, rk, bi, br)


def kernel(x, hidden, emb_table, kernel, rec_kernel, bias):
    xt_idx = x.T
    table2 = emb_table.reshape(VOCAB // 2, 2 * D)
    idx_half = (xt_idx >> 1).reshape(B * T)
    parity = (xt_idx & 1).astype(jnp.float32).reshape(T, B, 1)
    emb = _sc_gather(table2, idx_half)
    emb = emb.reshape(T, B, 2 * D)
    bi = bias[0].reshape(1, 3 * U)
    br = bias[1].reshape(1, 3 * U)
    outs, state = _tc_gru(emb, parity, hidden, kernel, rec_kernel, bi, br)
    return outs.reshape(B, T, U), state


# speedup vs baseline: 1.1645x; 1.1645x over previous
import jax
import jax.numpy as jnp
from jax.experimental import pallas as pl
from jax.experimental.pallas import tpu as pltpu
from jax.experimental.pallas import tpu_sc as plsc

VOCAB = 100000
D = 64
U = 128
B = 1024
T = 50

NCHUNK = 2
TC_ = T // NCHUNK

GATHER_WINDOW = 256


def _sc_gather(table2, idx_flat):
    n = idx_flat.shape[0]
    idx2 = idx_flat.reshape(1, n)
    mesh = plsc.VectorSubcoreMesh(core_axis_name="c", subcore_axis_name="s")

    @pl.kernel(
        out_type=jax.ShapeDtypeStruct((n, 2 * D), table2.dtype),
        mesh=mesh,
    )
    def gather_kernel(tab_hbm, i_hbm, o_hbm):
        def body(i_vmem, o_vmem):
            pltpu.sync_copy(tab_hbm.at[i_vmem.at[0]], o_vmem)

        pltpu.emit_pipeline(
            body,
            grid=(n // GATHER_WINDOW,),
            in_specs=[pl.BlockSpec((1, GATHER_WINDOW), lambda i: (0, i))],
            out_specs=[pl.BlockSpec((GATHER_WINDOW, 2 * D), lambda i: (i, 0))],
            core_axis_name=("c", "s"),
            dimension_semantics=(pltpu.PARALLEL,),
        )(i_hbm, o_hbm)

    return gather_kernel(table2, idx2)


def _gru_body(hid_ref, emb_ref, par_ref, w_ref, b_ref, out_ref,
              state_ref, h_ref):
    t = pl.program_id(0)

    @pl.when(t == 0)
    def _():
        h_ref[...] = hid_ref[...]

    g = emb_ref[0]
    par = par_ref[0]
    xt = jnp.where(par > 0.5, g[:, D:], g[:, :D])
    h = h_ref[...]
    hx = jnp.concatenate([h, xt], axis=1)
    cm = jnp.dot(hx, w_ref[...], preferred_element_type=jnp.float32)
    cm = cm + b_ref[...]
    z = jax.nn.sigmoid(cm[:, :U])
    r = jax.nn.sigmoid(cm[:, U:2 * U])
    hh = jnp.tanh(cm[:, 2 * U:3 * U] + r * cm[:, 3 * U:])
    h_new = z * h + (1.0 - z) * hh
    h_ref[...] = h_new
    out_ref[0] = h_new

    @pl.when(t == TC_ - 1)
    def _():
        state_ref[...] = h_new


def _combine_weights(k, rk, bias):
    zeros_u = jnp.zeros((U, U), jnp.float32)
    zeros_d = jnp.zeros((D, U), jnp.float32)
    w_h = jnp.concatenate([rk[:, :2 * U], zeros_u, rk[:, 2 * U:]], axis=1)
    w_x = jnp.concatenate([k[:, :2 * U], k[:, 2 * U:], zeros_d], axis=1)
    w = jnp.concatenate([w_h, w_x], axis=0)
    b = jnp.concatenate([bias[0, :2 * U] + bias[1, :2 * U],
                         bias[0, 2 * U:], bias[1, 2 * U:]]).reshape(1, 4 * U)
    return w, b


def _tc_gru(emb, parity, hidden, w, b):
    return pl.pallas_call(
        _gru_body,
        grid=(TC_,),
        in_specs=[
            pl.BlockSpec((B, U), lambda t: (0, 0)),
            pl.BlockSpec((1, B, 2 * D), lambda t: (t, 0, 0)),
            pl.BlockSpec((1, B, 1), lambda t: (t, 0, 0)),
            pl.BlockSpec((U + D, 4 * U), lambda t: (0, 0)),
            pl.BlockSpec((1, 4 * U), lambda t: (0, 0)),
        ],
        out_specs=[
            pl.BlockSpec((1, B, U), lambda t: (t, 0, 0)),
            pl.BlockSpec((B, U), lambda t: (0, 0)),
        ],
        out_shape=[
            jax.ShapeDtypeStruct((TC_, B, U), jnp.float32),
            jax.ShapeDtypeStruct((B, U), jnp.float32),
        ],
        scratch_shapes=[pltpu.VMEM((B, U), jnp.float32)],
        compiler_params=pltpu.CompilerParams(
            dimension_semantics=("arbitrary",),
        ),
    )(hidden, emb, parity, w, b)


def kernel(x, hidden, emb_table, kernel, rec_kernel, bias):
    xt_idx = x.T
    table2 = emb_table.reshape(VOCAB // 2, 2 * D)
    idx_half = (xt_idx >> 1).reshape(T, B)
    parity = (xt_idx & 1).astype(jnp.float32).reshape(T, B, 1)
    w, b = _combine_weights(kernel, rec_kernel, bias)

    embs = [
        _sc_gather(table2, idx_half[c * TC_:(c + 1) * TC_].reshape(TC_ * B))
        .reshape(TC_, B, 2 * D)
        for c in range(NCHUNK)
    ]

    outs = []
    state = hidden
    for c in range(NCHUNK):
        par_c = parity[c * TC_:(c + 1) * TC_]
        out_c, state = _tc_gru(embs[c], par_c, state, w, b)
        outs.append(out_c)
    outs = jnp.concatenate(outs, axis=0)
    return jnp.swapaxes(outs, 0, 1), state


# speedup vs baseline: 1.2848x; 1.1033x over previous
import jax
import jax.numpy as jnp
from jax.experimental import pallas as pl
from jax.experimental.pallas import tpu as pltpu
from jax.experimental.pallas import tpu_sc as plsc

VOCAB = 100000
D = 64
U = 128
B = 1024
T = 50

GATHER_WINDOW = 256


def _sc_gather(table2, idx_flat):
    n = idx_flat.shape[0]
    idx2 = idx_flat.reshape(1, n)
    mesh = plsc.VectorSubcoreMesh(core_axis_name="c", subcore_axis_name="s")

    @pl.kernel(
        out_type=jax.ShapeDtypeStruct((n, 2 * D), table2.dtype),
        mesh=mesh,
    )
    def gather_kernel(tab_hbm, i_hbm, o_hbm):
        def body(i_vmem, o_vmem):
            pltpu.sync_copy(tab_hbm.at[i_vmem.at[0]], o_vmem)

        pltpu.emit_pipeline(
            body,
            grid=(n // GATHER_WINDOW,),
            in_specs=[pl.BlockSpec((1, GATHER_WINDOW), lambda i: (0, i))],
            out_specs=[pl.BlockSpec((GATHER_WINDOW, 2 * D), lambda i: (i, 0))],
            core_axis_name=("c", "s"),
            dimension_semantics=(pltpu.PARALLEL,),
        )(i_hbm, o_hbm)

    return gather_kernel(table2, idx2)


def _gru_body(hid_ref, emb_ref, par_ref, w_ref, b_ref, out_ref,
              state_ref, h_ref):
    t = pl.program_id(0)

    @pl.when(t == 0)
    def _():
        h_ref[...] = hid_ref[...]

    g = emb_ref[0]
    par = par_ref[0]
    xt = jnp.where(par > 0.5, g[:, D:], g[:, :D])
    h = h_ref[...]
    hx = jnp.concatenate([h, xt], axis=1)
    cm = jnp.dot(hx, w_ref[...], preferred_element_type=jnp.float32)
    cm = cm + b_ref[...]
    z = jax.nn.sigmoid(cm[:, :U])
    r = jax.nn.sigmoid(cm[:, U:2 * U])
    hh = jnp.tanh(cm[:, 2 * U:3 * U] + r * cm[:, 3 * U:])
    h_new = z * h + (1.0 - z) * hh
    h_ref[...] = h_new
    out_ref[0] = h_new

    @pl.when(t == T - 1)
    def _():
        state_ref[...] = h_new


def _combine_weights(k, rk, bias):
    zeros_u = jnp.zeros((U, U), jnp.float32)
    zeros_d = jnp.zeros((D, U), jnp.float32)
    w_h = jnp.concatenate([rk[:, :2 * U], zeros_u, rk[:, 2 * U:]], axis=1)
    w_x = jnp.concatenate([k[:, :2 * U], k[:, 2 * U:], zeros_d], axis=1)
    w = jnp.concatenate([w_h, w_x], axis=0)
    b = jnp.concatenate([bias[0, :2 * U] + bias[1, :2 * U],
                         bias[0, 2 * U:], bias[1, 2 * U:]]).reshape(1, 4 * U)
    return w, b


def _tc_gru(emb, parity, hidden, w, b):
    return pl.pallas_call(
        _gru_body,
        grid=(T,),
        in_specs=[
            pl.BlockSpec((B, U), lambda t: (0, 0)),
            pl.BlockSpec((1, B, 2 * D), lambda t: (t, 0, 0)),
            pl.BlockSpec((1, B, 1), lambda t: (t, 0, 0)),
            pl.BlockSpec((U + D, 4 * U), lambda t: (0, 0)),
            pl.BlockSpec((1, 4 * U), lambda t: (0, 0)),
        ],
        out_specs=[
            pl.BlockSpec((1, B, U), lambda t: (t, 0, 0)),
            pl.BlockSpec((B, U), lambda t: (0, 0)),
        ],
        out_shape=[
            jax.ShapeDtypeStruct((T, B, U), jnp.float32),
            jax.ShapeDtypeStruct((B, U), jnp.float32),
        ],
        scratch_shapes=[pltpu.VMEM((B, U), jnp.float32)],
        compiler_params=pltpu.CompilerParams(
            dimension_semantics=("arbitrary",),
        ),
    )(hidden, emb, parity, w, b)


def kernel(x, hidden, emb_table, kernel, rec_kernel, bias):
    xt_idx = x.T
    table2 = emb_table.reshape(VOCAB // 2, 2 * D)
    idx_half = (xt_idx >> 1).reshape(B * T)
    parity = (xt_idx & 1).astype(jnp.float32).reshape(T, B, 1)
    emb = _sc_gather(table2, idx_half)
    emb = emb.reshape(T, B, 2 * D)
    w, b = _combine_weights(kernel, rec_kernel, bias)
    outs, state = _tc_gru(emb, parity, hidden, w, b)
    return jnp.swapaxes(outs, 0, 1), state
